# baseline (speedup 1.0000x reference)
import dataclasses
import functools

import jax
import jax.numpy as jnp
from jax.experimental import pallas as pl
from jax.experimental.pallas import tpu as pltpu
from jax.experimental.pallas import tpu_sc as plsc

N_ROWS = 1000000
N_COLS = 26
NS = 200
BR = 800
NB = N_ROWS // BR
BIAS = 4096
OFF2 = N_COLS - 16


def _sc_body(x_hbm, a_hbm, b_hbm, cb_hbm, yt_hbm, dy_hbm, o_hbm,
             yt_v, dy_v, a_v, b_v, cb_v):
    pltpu.sync_copy(yt_hbm, yt_v)
    pltpu.sync_copy(dy_hbm, dy_v)
    pltpu.sync_copy(a_hbm, a_v)
    pltpu.sync_copy(b_hbm, b_v)
    pltpu.sync_copy(cb_hbm, cb_v)

    def block(in_v, out_v):
        pats = [(a_v[pl.ds(v * 16, 16)], b_v[pl.ds(v * 16, 16)],
                 cb_v[pl.ds(v * 16, 16)]) for v in range(2)]

        @plsc.parallel_loop(0, BR, 1, unroll=4)
        def _(r):
            for v, (av, bv, cb) in enumerate(pats):
                off = v * OFF2
                x = in_v[r, pl.ds(off, 16)]
                fid = x * av + bv
                ti = fid.astype(jnp.int32)
                frac = fid - ti.astype(jnp.float32)
                gi = jnp.minimum(jnp.maximum(ti, BIAS), BIAS + NS - 2) + cb
                y0 = plsc.load_gather(yt_v, [gi])
                dy = plsc.load_gather(dy_v, [gi])
                out_v[r, pl.ds(off, 16)] = y0 + frac * dy

    pltpu.emit_pipeline(
        block,
        grid=(NB,),
        in_specs=[pl.BlockSpec((BR, N_COLS), lambda i: (i, 0))],
        out_specs=[pl.BlockSpec((BR, N_COLS), lambda i: (i, 0))],
        core_axis_name=("c", "s"),
        dimension_semantics=(pltpu.PARALLEL,),
    )(x_hbm, o_hbm)


def kernel(X, transforms_X, transforms_Y):
    xb0 = transforms_X[:, 0]
    xb1 = transforms_X[:, 1]
    rinv = 1.0 / (xb1 - xb0)
    a = rinv * (NS - 1)
    b = -xb0 * rinv * (NS - 1) + BIAS
    col = jnp.concatenate([jnp.arange(16, dtype=jnp.int32),
                           jnp.arange(OFF2, N_COLS, dtype=jnp.int32)])
    a_pat = a[col]
    b_pat = b[col]
    cb_pat = col * NS - BIAS
    ytab = transforms_Y.reshape(-1)
    dtab = jnp.pad(transforms_Y[:, 1:] - transforms_Y[:, :-1],
                   ((0, 0), (0, 1))).reshape(-1)
    mesh = plsc.VectorSubcoreMesh(core_axis_name="c", subcore_axis_name="s")
    cp = pltpu.CompilerParams()
    if "needs_layout_passes" in pltpu.CompilerParams.__dataclass_fields__:
        cp = dataclasses.replace(cp, needs_layout_passes=False)
    cp = dataclasses.replace(cp, use_tc_tiling_on_sc=False)

    run = functools.partial(
        pl.kernel,
        mesh=mesh,
        compiler_params=cp,
        out_type=jax.ShapeDtypeStruct((N_ROWS, N_COLS), jnp.float32),
        scratch_types=[
            pltpu.VMEM((N_COLS * NS,), jnp.float32),
            pltpu.VMEM((N_COLS * NS,), jnp.float32),
            pltpu.VMEM((32,), jnp.float32),
            pltpu.VMEM((32,), jnp.float32),
            pltpu.VMEM((32,), jnp.int32),
        ],
    )(_sc_body)

    return run(X, a_pat, b_pat, cb_pat, ytab, dtab)


# speedup vs baseline: 1.5761x; 1.5761x over previous
import dataclasses
import functools

import jax
import jax.numpy as jnp
from jax.experimental import pallas as pl
from jax.experimental.pallas import tpu as pltpu
from jax.experimental.pallas import tpu_sc as plsc

N_ROWS = 1000000
N_COLS = 26
NS = 200
BR = 200
NB = N_ROWS // BR
BIAS = 4096
OFF2 = N_COLS - 16


def _sc_body(x_hbm, a_hbm, b_hbm, cb_hbm, yt_hbm, dy_hbm, o_hbm,
             yt_v, dy_v, a_v, b_v, cb_v):
    pltpu.sync_copy(yt_hbm, yt_v)
    pltpu.sync_copy(dy_hbm, dy_v)
    pltpu.sync_copy(a_hbm, a_v)
    pltpu.sync_copy(b_hbm, b_v)
    pltpu.sync_copy(cb_hbm, cb_v)

    def block(in_v, out_v):
        pats = [(a_v[pl.ds(v * 16, 16)], b_v[pl.ds(v * 16, 16)],
                 cb_v[pl.ds(v * 16, 16)]) for v in range(2)]

        @plsc.parallel_loop(0, BR, 1, unroll=4)
        def _(r):
            for v, (av, bv, cb) in enumerate(pats):
                off = v * OFF2
                x = in_v[r, pl.ds(off, 16)]
                fid = x * av + bv
                ti = fid.astype(jnp.int32)
                frac = fid - ti.astype(jnp.float32)
                gi = jnp.minimum(jnp.maximum(ti, BIAS), BIAS + NS - 2) + cb
                y0 = plsc.load_gather(yt_v, [gi])
                dy = plsc.load_gather(dy_v, [gi])
                out_v[r, pl.ds(off, 16)] = y0 + frac * dy

    pltpu.emit_pipeline(
        block,
        grid=(NB,),
        in_specs=[pl.BlockSpec((BR, N_COLS), lambda i: (i, 0))],
        out_specs=[pl.BlockSpec((BR, N_COLS), lambda i: (i, 0))],
        core_axis_name=("c", "s"),
        dimension_semantics=(pltpu.PARALLEL,),
    )(x_hbm, o_hbm)


def kernel(X, transforms_X, transforms_Y):
    xb0 = transforms_X[:, 0]
    xb1 = transforms_X[:, 1]
    rinv = 1.0 / (xb1 - xb0)
    a = rinv * (NS - 1)
    b = -xb0 * rinv * (NS - 1) + BIAS
    col = jnp.concatenate([jnp.arange(16, dtype=jnp.int32),
                           jnp.arange(OFF2, N_COLS, dtype=jnp.int32)])
    a_pat = a[col]
    b_pat = b[col]
    cb_pat = col * NS - BIAS
    ytab = transforms_Y.reshape(-1)
    dtab = jnp.pad(transforms_Y[:, 1:] - transforms_Y[:, :-1],
                   ((0, 0), (0, 1))).reshape(-1)
    mesh = plsc.VectorSubcoreMesh(core_axis_name="c", subcore_axis_name="s")
    cp = pltpu.CompilerParams()
    if "needs_layout_passes" in pltpu.CompilerParams.__dataclass_fields__:
        cp = dataclasses.replace(cp, needs_layout_passes=False)

    run = functools.partial(
        pl.kernel,
        mesh=mesh,
        compiler_params=cp,
        out_type=jax.ShapeDtypeStruct((N_ROWS, N_COLS), jnp.float32),
        scratch_types=[
            pltpu.VMEM((N_COLS * NS,), jnp.float32),
            pltpu.VMEM((N_COLS * NS,), jnp.float32),
            pltpu.VMEM((32,), jnp.float32),
            pltpu.VMEM((32,), jnp.float32),
            pltpu.VMEM((32,), jnp.int32),
        ],
    )(_sc_body)

    return run(X, a_pat, b_pat, cb_pat, ytab, dtab)
